# Initial kernel scaffold; baseline (speedup 1.0000x reference)
import jax
import jax.numpy as jnp
from jax.experimental import pallas as pl


def kernel(vert_feat_in, vert_mask_in, vert_element_oh, adj_oh, atom_subsets, atom_subsets_peaks, ln1_g, ln1_b, W1, b1, W2, b2, ln2_g, ln2_b, Ws, bs):
    raise NotImplementedError("write your pallas kernel here")


# speedup vs baseline: 4.1722x; 4.1722x over previous
import functools

import jax
import jax.numpy as jnp
from jax import lax
from jax.experimental import pallas as pl
from jax.experimental.pallas import tpu as pltpu
from jax.experimental.pallas import tpu_sc as plsc

_B, _A, _GF, _S, _P, _BINS, _D = 32, 32, 64, 4096, 8, 512, 128
_NPS = _P * _S
_LANES = 16
_LN_EPS = 1e-5


def _tc_body(mask_ref, mvf_ref, subs_ref, inten_ref, ln1g_ref, ln1b_ref,
             w1t_ref, b1_ref, w2t_ref, b2_ref, ln2g_ref, ln2b_ref,
             ws_ref, bs_ref, probs_ref, vals_ref):
    subs = subs_ref[0].astype(jnp.float32) * mask_ref[0]
    ssum = jnp.dot(subs, mvf_ref[0], preferred_element_type=jnp.float32)
    size = jnp.sum(subs, axis=1, keepdims=True) + 0.0001
    mean = ssum / size
    mu = jnp.mean(mean, axis=1, keepdims=True)
    var = jnp.mean((mean - mu) ** 2, axis=1, keepdims=True)
    xn = (mean - mu) * lax.rsqrt(var + _LN_EPS) * ln1g_ref[0] + ln1b_ref[0]
    x = jnp.maximum(
        jnp.dot(xn, w1t_ref[...], preferred_element_type=jnp.float32)
        + b1_ref[0], 0.0)
    x = jnp.maximum(
        jnp.dot(x, w2t_ref[...], preferred_element_type=jnp.float32)
        + b2_ref[0], 0.0)
    mu2 = jnp.mean(x, axis=1, keepdims=True)
    var2 = jnp.mean((x - mu2) ** 2, axis=1, keepdims=True)
    x2 = (x - mu2) * lax.rsqrt(var2 + _LN_EPS) * ln2g_ref[0] + ln2b_ref[0]
    scores = jnp.sum(x2 * ws_ref[0], axis=1) + bs_ref[0, 0]
    m = jnp.max(scores)
    e = jnp.exp(scores - m)
    probs = e / jnp.sum(e)
    probs_ref[0] = probs[None, :]
    vals_ref[0] = inten_ref[0] * probs[None, :]


def _tc_dense(mask3, mvf, subs, inten_t, ln1g, ln1b, w1t, b1, w2t, b2,
              ln2g, ln2b, ws, bs):
    grid = (_B,)
    in_specs = [
        pl.BlockSpec((1, 1, _A), lambda b: (b, 0, 0)),
        pl.BlockSpec((1, _A, _GF), lambda b: (b, 0, 0)),
        pl.BlockSpec((1, _S, _A), lambda b: (b, 0, 0)),
        pl.BlockSpec((1, _P, _S), lambda b: (b, 0, 0)),
        pl.BlockSpec((1, _GF), lambda b: (0, 0)),
        pl.BlockSpec((1, _GF), lambda b: (0, 0)),
        pl.BlockSpec((_GF, _D), lambda b: (0, 0)),
        pl.BlockSpec((1, _D), lambda b: (0, 0)),
        pl.BlockSpec((_D, _D), lambda b: (0, 0)),
        pl.BlockSpec((1, _D), lambda b: (0, 0)),
        pl.BlockSpec((1, _D), lambda b: (0, 0)),
        pl.BlockSpec((1, _D), lambda b: (0, 0)),
        pl.BlockSpec((1, _D), lambda b: (0, 0)),
        pl.BlockSpec((1, 1), lambda b: (0, 0)),
    ]
    out_specs = [
        pl.BlockSpec((1, 1, _S), lambda b: (b, 0, 0)),
        pl.BlockSpec((1, _P, _S), lambda b: (b, 0, 0)),
    ]
    out_shape = [
        jax.ShapeDtypeStruct((_B, 1, _S), jnp.float32),
        jax.ShapeDtypeStruct((_B, _P, _S), jnp.float32),
    ]
    return pl.pallas_call(
        _tc_body,
        grid=grid,
        in_specs=in_specs,
        out_specs=out_specs,
        out_shape=out_shape,
        compiler_params=pltpu.CompilerParams(
            dimension_semantics=("arbitrary",)),
    )(mask3, mvf, subs, inten_t, ln1g, ln1b, w1t, b1, w2t, b2,
      ln2g, ln2b, ws, bs)


def _sc_body(mass_hbm, vals_hbm, out_hbm, mass_v, vals_v, hist_v, row_v):
    w = lax.axis_index("s") * 2 + lax.axis_index("c")
    pltpu.sync_copy(mass_hbm.at[w], mass_v)
    pltpu.sync_copy(vals_hbm.at[w], vals_v)

    zeros16 = jnp.zeros((_LANES,), jnp.float32)

    def zero_body(i, carry):
        hist_v[pl.ds(i * _LANES, _LANES)] = zeros16
        return carry

    lax.fori_loop(0, (_LANES * _BINS) // _LANES, zero_body, 0)

    base = lax.iota(jnp.int32, _LANES) * _BINS

    def body(i, carry):
        mf = mass_v[pl.ds(i * _LANES, _LANES)]
        mf = jnp.minimum(jnp.maximum(mf, 0.0), 511.0)
        idx = (mf + 0.5).astype(jnp.int32) + base
        v = vals_v[pl.ds(i * _LANES, _LANES)]
        plsc.addupdate_scatter(hist_v, [idx], v)
        return carry

    lax.fori_loop(0, _NPS // _LANES, body, 0)

    def red_body(c, carry):
        acc = jnp.zeros((_LANES,), jnp.float32)
        for lane in range(_LANES):
            acc = acc + hist_v[pl.ds(lane * _BINS + c * _LANES, _LANES)]
        row_v[pl.ds(c * _LANES, _LANES)] = acc
        return carry

    lax.fori_loop(0, _BINS // _LANES, red_body, 0)
    pltpu.sync_copy(row_v, out_hbm.at[w])


def _sc_hist(mass2d, vals2d):
    mesh = plsc.VectorSubcoreMesh(core_axis_name="c", subcore_axis_name="s")
    f = functools.partial(
        pl.kernel,
        out_type=jax.ShapeDtypeStruct((_B, _BINS), jnp.float32),
        mesh=mesh,
        scratch_types=[
            pltpu.VMEM((_NPS,), jnp.float32),
            pltpu.VMEM((_NPS,), jnp.float32),
            pltpu.VMEM((_LANES * _BINS,), jnp.float32),
            pltpu.VMEM((_BINS,), jnp.float32),
        ],
        compiler_params=pltpu.CompilerParams(needs_layout_passes=False),
    )(_sc_body)
    return f(mass2d, vals2d)


def kernel(vert_feat_in, vert_mask_in, vert_element_oh, adj_oh, atom_subsets,
           atom_subsets_peaks, ln1_g, ln1_b, W1, b1, W2, b2, ln2_g, ln2_b,
           Ws, bs):
    mvf = vert_feat_in * vert_mask_in[..., None]
    mask3 = vert_mask_in[:, None, :]
    peaks_t = jnp.transpose(atom_subsets_peaks, (0, 3, 2, 1))
    mass_t = peaks_t[:, 0].reshape(_B, _NPS)
    inten_t = peaks_t[:, 1]

    probs, vals = _tc_dense(
        mask3, mvf, atom_subsets, inten_t,
        ln1_g.reshape(1, _GF), ln1_b.reshape(1, _GF),
        W1.T, b1.reshape(1, _D), W2.T, b2.reshape(1, _D),
        ln2_g.reshape(1, _D), ln2_b.reshape(1, _D),
        Ws.reshape(1, _D), bs.reshape(1, 1))

    spect = _sc_hist(mass_t, vals.reshape(_B, _NPS))
    return (spect, probs.reshape(_B, _S))


# speedup vs baseline: 6.5277x; 1.5646x over previous
import functools

import jax
import jax.numpy as jnp
from jax import lax
from jax.experimental import pallas as pl
from jax.experimental.pallas import tpu as pltpu
from jax.experimental.pallas import tpu_sc as plsc

_B, _A, _GF, _S, _P, _BINS, _D = 32, 32, 64, 4096, 8, 512, 128
_NPS = _P * _S
_LANES = 16
_LN_EPS = 1e-5


def _tc_body(mvfte_ref, subs_ref, m1_ref, w1c_ref, w2c_ref, wsg2_ref,
             m2r_ref, swg_ref, c2s_ref, probs_ref):
    subs = subs_ref[0].astype(jnp.float32)
    full = lax.dot_general(mvfte_ref[0], subs, (((1,), (1,)), ((), ())),
                           preferred_element_type=jnp.float32)
    mu = jnp.dot(m1_ref[...], full,
                 preferred_element_type=jnp.float32)
    musq = jnp.dot(m1_ref[...], full * full,
                   preferred_element_type=jnp.float32)
    var = musq - mu * mu
    size = full[_GF:_GF + 1]
    r = lax.rsqrt(var + _LN_EPS * size * size)
    t = (full[:_GF] - mu) * r
    t_ext = jnp.concatenate([t, jnp.ones((8, _S), jnp.float32)], axis=0)
    x = jnp.maximum(jnp.dot(w1c_ref[...], t_ext,
                            preferred_element_type=jnp.float32), 0.0)
    x_ext = jnp.concatenate([x, jnp.ones((8, _S), jnp.float32)], axis=0)
    x = jnp.maximum(jnp.dot(w2c_ref[...], x_ext,
                            preferred_element_type=jnp.float32), 0.0)
    u = jnp.dot(wsg2_ref[...], x, preferred_element_type=jnp.float32)
    mu2 = jnp.dot(m2r_ref[...], x, preferred_element_type=jnp.float32)
    musq2 = jnp.dot(m2r_ref[...], x * x,
                    preferred_element_type=jnp.float32)
    var2 = musq2 - mu2 * mu2
    r2 = lax.rsqrt(var2 + _LN_EPS)
    scores = r2 * (u - mu2 * swg_ref[0, 0]) + c2s_ref[0, 0]
    m = jnp.max(scores)
    e = jnp.exp(scores - m)
    probs_ref[0] = e / jnp.sum(e)


def _tc_dense(mvfte, subs, m1, w1c, w2c, wsg2, m2r, swg, c2s):
    grid = (_B,)
    in_specs = [
        pl.BlockSpec((1, 72, _A), lambda b: (b, 0, 0)),
        pl.BlockSpec((1, _S, _A), lambda b: (b, 0, 0)),
        pl.BlockSpec((1, 72), lambda b: (0, 0)),
        pl.BlockSpec((_D, 72), lambda b: (0, 0)),
        pl.BlockSpec((_D, _D + 8), lambda b: (0, 0)),
        pl.BlockSpec((1, _D), lambda b: (0, 0)),
        pl.BlockSpec((1, _D), lambda b: (0, 0)),
        pl.BlockSpec((1, 1), lambda b: (0, 0)),
        pl.BlockSpec((1, 1), lambda b: (0, 0)),
    ]
    out_specs = pl.BlockSpec((1, 1, _S), lambda b: (b, 0, 0))
    out_shape = jax.ShapeDtypeStruct((_B, 1, _S), jnp.float32)
    return pl.pallas_call(
        _tc_body,
        grid=grid,
        in_specs=in_specs,
        out_specs=out_specs,
        out_shape=out_shape,
        compiler_params=pltpu.CompilerParams(
            dimension_semantics=("arbitrary",)),
    )(mvfte, subs, m1, w1c, w2c, wsg2, m2r, swg, c2s)


def _sc_body(peaks_hbm, probs_hbm, out_hbm, peaks_v, probs_v, hist_v, row_v):
    w = lax.axis_index("s") * 2 + lax.axis_index("c")
    pltpu.sync_copy(peaks_hbm.at[w], peaks_v)
    pltpu.sync_copy(probs_hbm.at[w], probs_v)

    zeros16 = jnp.zeros((_LANES,), jnp.float32)

    def zero_body(i, carry):
        hist_v[pl.ds(i * _LANES, _LANES)] = zeros16
        return carry

    lax.fori_loop(0, (_LANES * _BINS) // _LANES, zero_body, 0)

    lane = lax.iota(jnp.int32, _LANES)
    base = lane * _BINS

    def body(i, carry):
        pair = i * _LANES + lane
        mf = plsc.load_gather(peaks_v, [pair * 2])
        iv = plsc.load_gather(peaks_v, [pair * 2 + 1])
        pr = plsc.load_gather(probs_v, [lax.shift_right_logical(pair, 3)])
        mf = jnp.minimum(jnp.maximum(mf, 0.0), 511.0)
        idx = (mf + 0.5).astype(jnp.int32) + base
        plsc.addupdate_scatter(hist_v, [idx], iv * pr)
        return carry

    lax.fori_loop(0, _NPS // _LANES, body, 0)

    def red_body(c, carry):
        acc = jnp.zeros((_LANES,), jnp.float32)
        for l in range(_LANES):
            acc = acc + hist_v[pl.ds(l * _BINS + c * _LANES, _LANES)]
        row_v[pl.ds(c * _LANES, _LANES)] = acc
        return carry

    lax.fori_loop(0, _BINS // _LANES, red_body, 0)
    pltpu.sync_copy(row_v, out_hbm.at[w])


def _sc_hist(peaks_flat, probs2d):
    mesh = plsc.VectorSubcoreMesh(core_axis_name="c", subcore_axis_name="s")
    f = functools.partial(
        pl.kernel,
        out_type=jax.ShapeDtypeStruct((_B, _BINS), jnp.float32),
        mesh=mesh,
        scratch_types=[
            pltpu.VMEM((2 * _NPS,), jnp.float32),
            pltpu.VMEM((_S,), jnp.float32),
            pltpu.VMEM((_LANES * _BINS,), jnp.float32),
            pltpu.VMEM((_BINS,), jnp.float32),
        ],
        compiler_params=pltpu.CompilerParams(needs_layout_passes=False),
    )(_sc_body)
    return f(peaks_flat, probs2d)


def kernel(vert_feat_in, vert_mask_in, vert_element_oh, adj_oh, atom_subsets,
           atom_subsets_peaks, ln1_g, ln1_b, W1, b1, W2, b2, ln2_g, ln2_b,
           Ws, bs):
    m2 = vert_mask_in * vert_mask_in
    top = jnp.transpose(vert_feat_in * m2[..., None], (0, 2, 1))
    mid = vert_mask_in[:, None, :]
    pad = jnp.zeros((_B, 72 - _GF - 1, _A), jnp.float32)
    mvfte = jnp.concatenate([top, mid, pad], axis=1)

    m1 = jnp.concatenate(
        [jnp.full((_GF,), 1.0 / _GF, jnp.float32),
         jnp.zeros((72 - _GF,), jnp.float32)]).reshape(1, 72)
    w1g = W1 * ln1_g[None, :]
    c1 = (W1 @ ln1_b + b1)[:, None]
    w1c = jnp.concatenate([w1g, c1, jnp.zeros((_D, 7), jnp.float32)], axis=1)
    w2c = jnp.concatenate([W2, b2[:, None], jnp.zeros((_D, 7), jnp.float32)],
                          axis=1)
    wsg2 = (Ws[0] * ln2_g).reshape(1, _D)
    m2r = jnp.full((1, _D), 1.0 / _D, jnp.float32)
    swg = jnp.sum(Ws[0] * ln2_g).reshape(1, 1)
    c2s = (jnp.sum(Ws[0] * ln2_b) + bs[0]).reshape(1, 1)

    peaks_flat = atom_subsets_peaks.reshape(_B, 2 * _NPS)

    probs = _tc_dense(mvfte, atom_subsets, m1, w1c, w2c, wsg2, m2r, swg, c2s)

    probs2d = probs.reshape(_B, _S)
    spect = _sc_hist(peaks_flat, probs2d)
    return (spect, probs2d)


# speedup vs baseline: 12.8259x; 1.9649x over previous
import functools

import jax
import jax.numpy as jnp
from jax import lax
from jax.experimental import pallas as pl
from jax.experimental.pallas import tpu as pltpu
from jax.experimental.pallas import tpu_sc as plsc

_B, _A, _GF, _S, _P, _BINS, _D = 32, 32, 64, 4096, 8, 512, 128
_NPS = _P * _S
_LANES = 16
_LN_EPS = 1e-5


def _tc_body(mvfte_ref, subs_ref, m1_ref, w1c_ref, w2c_ref, wsg2_ref,
             m2r_ref, swg_ref, c2s_ref, probs_ref):
    subs = subs_ref[0].astype(jnp.float32)
    full = jnp.dot(mvfte_ref[0], subs, preferred_element_type=jnp.float32)
    mu = jnp.dot(m1_ref[...], full,
                 preferred_element_type=jnp.float32)
    musq = jnp.dot(m1_ref[...], full * full,
                   preferred_element_type=jnp.float32)
    var = musq - mu * mu
    size = full[_GF:_GF + 1]
    r = lax.rsqrt(var + _LN_EPS * size * size)
    t = (full[:_GF] - mu) * r
    t_ext = jnp.concatenate([t, jnp.ones((8, _S), jnp.float32)], axis=0)
    x = jnp.maximum(jnp.dot(w1c_ref[...], t_ext,
                            preferred_element_type=jnp.float32), 0.0)
    x_ext = jnp.concatenate([x, jnp.ones((8, _S), jnp.float32)], axis=0)
    x = jnp.maximum(jnp.dot(w2c_ref[...], x_ext,
                            preferred_element_type=jnp.float32), 0.0)
    u = jnp.dot(wsg2_ref[...], x, preferred_element_type=jnp.float32)
    mu2 = jnp.dot(m2r_ref[...], x, preferred_element_type=jnp.float32)
    musq2 = jnp.dot(m2r_ref[...], x * x,
                    preferred_element_type=jnp.float32)
    var2 = musq2 - mu2 * mu2
    r2 = lax.rsqrt(var2 + _LN_EPS)
    scores = r2 * (u - mu2 * swg_ref[0, 0]) + c2s_ref[0, 0]
    m = jnp.max(scores)
    e = jnp.exp(scores - m)
    probs_ref[0] = e / jnp.sum(e)


def _tc_dense(mvfte, subs, m1, w1c, w2c, wsg2, m2r, swg, c2s):
    grid = (_B,)
    in_specs = [
        pl.BlockSpec((1, 72, _A), lambda b: (b, 0, 0)),
        pl.BlockSpec((1, _A, _S), lambda b: (b, 0, 0)),
        pl.BlockSpec((1, 72), lambda b: (0, 0)),
        pl.BlockSpec((_D, 72), lambda b: (0, 0)),
        pl.BlockSpec((_D, _D + 8), lambda b: (0, 0)),
        pl.BlockSpec((1, _D), lambda b: (0, 0)),
        pl.BlockSpec((1, _D), lambda b: (0, 0)),
        pl.BlockSpec((1, 1), lambda b: (0, 0)),
        pl.BlockSpec((1, 1), lambda b: (0, 0)),
    ]
    out_specs = pl.BlockSpec((1, 1, _S), lambda b: (b, 0, 0))
    out_shape = jax.ShapeDtypeStruct((_B, 1, _S), jnp.float32)
    return pl.pallas_call(
        _tc_body,
        grid=grid,
        in_specs=in_specs,
        out_specs=out_specs,
        out_shape=out_shape,
        compiler_params=pltpu.CompilerParams(
            dimension_semantics=("arbitrary",)),
    )(mvfte, subs, m1, w1c, w2c, wsg2, m2r, swg, c2s)


def _sc_body(peaks_hbm, probs_hbm, out_hbm, peaks_v, probs_v, hist_v, row_v):
    w = lax.axis_index("s") * 2 + lax.axis_index("c")
    pltpu.sync_copy(peaks_hbm.at[w], peaks_v)
    pltpu.sync_copy(probs_hbm.at[w], probs_v)

    zeros16 = jnp.zeros((_LANES,), jnp.float32)

    @plsc.parallel_loop(0, (_LANES * _BINS) // _LANES, 1, unroll=8)
    def _zero(i):
        hist_v[pl.ds(i * _LANES, _LANES)] = zeros16

    base = lax.iota(jnp.int32, _LANES) * _BINS

    @plsc.parallel_loop(0, _S // _LANES, 1, unroll=2)
    def _scatter(i):
        s0 = i * _LANES
        pr = probs_v[pl.ds(s0, _LANES)]
        row = (i >> 3) * 2
        col = (i & 7) * _LANES
        for p in range(_P):
            mf = peaks_v[p * 64 + row, pl.ds(col, _LANES)]
            iv = peaks_v[p * 64 + row + 1, pl.ds(col, _LANES)]
            mf = jnp.minimum(jnp.maximum(mf, 0.0), 511.0)
            idx = (mf + 0.5).astype(jnp.int32) + base
            plsc.addupdate_scatter(hist_v, [idx], iv * pr)

    @plsc.parallel_loop(0, _BINS // _LANES, 1, unroll=2)
    def _reduce(c):
        acc = hist_v[pl.ds(c * _LANES, _LANES)]
        for l in range(1, _LANES):
            acc = acc + hist_v[pl.ds(l * _BINS + c * _LANES, _LANES)]
        row_v[pl.ds(c * _LANES, _LANES)] = acc

    pltpu.sync_copy(row_v, out_hbm.at[w])


def _sc_hist(peaks_flat, probs2d):
    mesh = plsc.VectorSubcoreMesh(core_axis_name="c", subcore_axis_name="s")
    f = functools.partial(
        pl.kernel,
        out_type=jax.ShapeDtypeStruct((_B, _BINS), jnp.float32),
        mesh=mesh,
        scratch_types=[
            pltpu.VMEM((2 * _NPS // 128, 128), jnp.float32),
            pltpu.VMEM((_S,), jnp.float32),
            pltpu.VMEM((_LANES * _BINS,), jnp.float32),
            pltpu.VMEM((_BINS,), jnp.float32),
        ],
        compiler_params=pltpu.CompilerParams(needs_layout_passes=False),
    )(_sc_body)
    return f(peaks_flat, probs2d)


def kernel(vert_feat_in, vert_mask_in, vert_element_oh, adj_oh, atom_subsets,
           atom_subsets_peaks, ln1_g, ln1_b, W1, b1, W2, b2, ln2_g, ln2_b,
           Ws, bs):
    m2 = vert_mask_in * vert_mask_in
    top = jnp.transpose(vert_feat_in * m2[..., None], (0, 2, 1))
    mid = vert_mask_in[:, None, :]
    pad = jnp.zeros((_B, 72 - _GF - 1, _A), jnp.float32)
    mvfte = jnp.concatenate([top, mid, pad], axis=1)

    m1 = jnp.concatenate(
        [jnp.full((_GF,), 1.0 / _GF, jnp.float32),
         jnp.zeros((72 - _GF,), jnp.float32)]).reshape(1, 72)
    w1g = W1 * ln1_g[None, :]
    c1 = (W1 @ ln1_b + b1)[:, None]
    w1c = jnp.concatenate([w1g, c1, jnp.zeros((_D, 7), jnp.float32)], axis=1)
    w2c = jnp.concatenate([W2, b2[:, None], jnp.zeros((_D, 7), jnp.float32)],
                          axis=1)
    wsg2 = (Ws[0] * ln2_g).reshape(1, _D)
    m2r = jnp.full((1, _D), 1.0 / _D, jnp.float32)
    swg = jnp.sum(Ws[0] * ln2_g).reshape(1, 1)
    c2s = (jnp.sum(Ws[0] * ln2_b) + bs[0]).reshape(1, 1)

    peaks_flat = jnp.transpose(
        jnp.transpose(atom_subsets_peaks, (0, 2, 3, 1))
        .reshape(_B, _P, 2, _S // 128, 128),
        (0, 1, 3, 2, 4)).reshape(_B, 2 * _NPS // 128, 128)

    subs_t = jnp.transpose(atom_subsets, (0, 2, 1))
    probs = _tc_dense(mvfte, subs_t, m1, w1c, w2c, wsg2, m2r, swg, c2s)

    probs2d = probs.reshape(_B, _S)
    spect = _sc_hist(peaks_flat, probs2d)
    return (spect, probs2d)
